# Initial kernel scaffold; baseline (speedup 1.0000x reference)
import jax
import jax.numpy as jnp
from jax.experimental import pallas as pl


def kernel(x, tables, W, b):
    raise NotImplementedError("write your pallas kernel here")


# speedup vs baseline: 1.6085x; 1.6085x over previous
import functools

import jax
import jax.numpy as jnp
from jax import lax
from jax.experimental import pallas as pl
from jax.experimental.pallas import tpu as pltpu
from jax.experimental.pallas import tpu_sc as plsc

NUM_FIELDS = 26
VOCAB = 100000
HIDDEN = 64
BATCH = 16384

BF = BATCH * NUM_FIELDS
CHUNK = 128
NC = 2
NS = 16
NW = NC * NS
N_CHUNKS = BF // CHUNK
CPW = N_CHUNKS // NW

_MESH = plsc.VectorSubcoreMesh(core_axis_name="c", subcore_axis_name="s")


@functools.partial(
    pl.kernel,
    mesh=_MESH,
    out_type=jax.ShapeDtypeStruct((BF, HIDDEN), jnp.float32),
    scratch_types=[
        pltpu.VMEM((CPW, CHUNK), jnp.int32),
        pltpu.VMEM((CHUNK, HIDDEN), jnp.float32),
        pltpu.SemaphoreType.DMA,
    ],
    compiler_params=pltpu.CompilerParams(use_tc_tiling_on_sc=False),
)
def _sc_gather(tab_hbm, idx_hbm, out_hbm, idx_v, rows_v, gsem):
    wid = lax.axis_index("s") * NC + lax.axis_index("c")
    cbase = wid * CPW
    pltpu.sync_copy(idx_hbm.at[pl.ds(cbase, CPW)], idx_v)

    def body(j, carry):
        pltpu.async_copy(tab_hbm.at[idx_v.at[j]], rows_v, gsem).wait()
        pltpu.sync_copy(rows_v, out_hbm.at[pl.ds((cbase + j) * CHUNK, CHUNK)])
        return carry

    lax.fori_loop(0, CPW, body, 0)


def _mm_body(g_ref, w_ref, b_ref, o_ref):
    o_ref[...] = (
        jnp.dot(g_ref[...], w_ref[...], preferred_element_type=jnp.float32)
        + b_ref[...]
    )


def _tc_project(gathered, W, b):
    BM = 512
    FH = NUM_FIELDS * HIDDEN
    return pl.pallas_call(
        _mm_body,
        grid=(BATCH // BM,),
        in_specs=[
            pl.BlockSpec((BM, FH), lambda i: (i, 0)),
            pl.BlockSpec((FH, HIDDEN), lambda i: (0, 0)),
            pl.BlockSpec((1, HIDDEN), lambda i: (0, 0)),
        ],
        out_specs=pl.BlockSpec((BM, HIDDEN), lambda i: (i, 0)),
        out_shape=jax.ShapeDtypeStruct((BATCH, HIDDEN), jnp.float32),
    )(gathered, W, b.reshape(1, HIDDEN))


def kernel(x, tables, W, b):
    offs = jnp.arange(NUM_FIELDS, dtype=jnp.int32) * VOCAB
    idx = (x.astype(jnp.int32) + offs[None, :]).reshape(N_CHUNKS, CHUNK)
    tab_flat = tables.reshape(NUM_FIELDS * VOCAB, HIDDEN)
    gathered = _sc_gather(tab_flat, idx)
    return _tc_project(gathered.reshape(BATCH, NUM_FIELDS * HIDDEN), W, b)


# speedup vs baseline: 1.7051x; 1.0601x over previous
import functools

import jax
import jax.numpy as jnp
from jax import lax
from jax.experimental import pallas as pl
from jax.experimental.pallas import tpu as pltpu
from jax.experimental.pallas import tpu_sc as plsc

NUM_FIELDS = 26
VOCAB = 100000
HIDDEN = 64
BATCH = 16384

BF = BATCH * NUM_FIELDS
CHUNK = 128
NC = 2
NS = 16
NW = NC * NS
N_CHUNKS = BF // CHUNK
CPW = N_CHUNKS // NW

_MESH = plsc.VectorSubcoreMesh(core_axis_name="c", subcore_axis_name="s")


@functools.partial(
    pl.kernel,
    mesh=_MESH,
    out_type=jax.ShapeDtypeStruct((BF, HIDDEN), jnp.float32),
    scratch_types=[
        pltpu.VMEM((CPW, CHUNK), jnp.int32),
        pltpu.VMEM((CHUNK, 2 * HIDDEN), jnp.float32),
        pltpu.SemaphoreType.DMA,
    ],
    compiler_params=pltpu.CompilerParams(use_tc_tiling_on_sc=False),
)
def _sc_gather(tab_hbm, idx_hbm, out_hbm, idx_v, rows_v, gsem):
    wid = lax.axis_index("s") * NC + lax.axis_index("c")
    cbase = wid * CPW
    pltpu.sync_copy(idx_hbm.at[pl.ds(cbase, CPW)], idx_v)

    def body(j, carry):
        pltpu.async_copy(tab_hbm.at[idx_v.at[j]], rows_v, gsem).wait()
        pltpu.sync_copy(rows_v.at[:, pl.ds(0, HIDDEN)],
                        out_hbm.at[pl.ds((cbase + j) * CHUNK, CHUNK)])
        return carry

    lax.fori_loop(0, CPW, body, 0)


def _mm_body(g_ref, w_ref, b_ref, o_ref):
    o_ref[...] = (
        jnp.dot(g_ref[...], w_ref[...], preferred_element_type=jnp.float32)
        + b_ref[...]
    )


def _tc_project(gathered, W, b):
    BM = 512
    FH = NUM_FIELDS * HIDDEN
    return pl.pallas_call(
        _mm_body,
        grid=(BATCH // BM,),
        in_specs=[
            pl.BlockSpec((BM, FH), lambda i: (i, 0)),
            pl.BlockSpec((FH, HIDDEN), lambda i: (0, 0)),
            pl.BlockSpec((1, HIDDEN), lambda i: (0, 0)),
        ],
        out_specs=pl.BlockSpec((BM, HIDDEN), lambda i: (i, 0)),
        out_shape=jax.ShapeDtypeStruct((BATCH, HIDDEN), jnp.float32),
    )(gathered, W, b.reshape(1, HIDDEN))


def kernel(x, tables, W, b):
    offs = jnp.arange(NUM_FIELDS, dtype=jnp.int32) * VOCAB
    idx = (x.astype(jnp.int32) + offs[None, :]).reshape(N_CHUNKS, CHUNK)
    tab_flat = jnp.pad(tables, ((0, 0), (0, 0), (0, HIDDEN))).reshape(
        NUM_FIELDS * VOCAB, 2 * HIDDEN)
    gathered = _sc_gather(tab_flat, idx)
    return _tc_project(gathered.reshape(BATCH, NUM_FIELDS * HIDDEN), W, b)
